# Initial kernel scaffold; baseline (speedup 1.0000x reference)
import jax
import jax.numpy as jnp
from jax.experimental import pallas as pl


def kernel(xb):
    raise NotImplementedError("write your pallas kernel here")


# speedup vs baseline: 1.3545x; 1.3545x over previous
import functools

import jax
import jax.numpy as jnp
from jax import lax
from jax.experimental import pallas as pl


def _mask_body(len_keep, l_block, nrow_ref, ncol_ref, xb_ref, out_ref, mask_ref):
    L = nrow_ref.shape[-1]
    nvars = mask_ref.shape[-1]
    nj = nrow_ref[0]
    nl = ncol_ref[0]
    jidx = lax.broadcasted_iota(jnp.int32, (l_block, L), 1)
    lg = pl.program_id(1) * l_block + lax.broadcasted_iota(
        jnp.int32, (l_block, L), 0)
    cnt = (nj < nl) | ((nj == nl) & (jidx < lg))
    rank = jnp.sum(cnt.astype(jnp.int32), axis=1, keepdims=True)
    keep = (rank < len_keep).astype(jnp.float32)
    out_ref[0] = xb_ref[0] * keep
    mask_ref[0] = jnp.broadcast_to(1.0 - keep, (l_block, nvars))


@jax.jit
def kernel(xb):
    bs, L, nvars, D = xb.shape
    len_keep = int(L * (1 - 0.15))
    noise = jax.random.uniform(jax.random.key(42), (bs, L), dtype=jnp.float32)
    nrow = noise.reshape(bs, 1, L)
    ncol = noise.reshape(bs, L, 1)
    x2 = xb.reshape(bs, L, nvars * D)

    l_block = 64
    grid = (bs, L // l_block)
    x_masked, mask = pl.pallas_call(
        functools.partial(_mask_body, len_keep, l_block),
        grid=grid,
        in_specs=[
            pl.BlockSpec((1, 1, L), lambda b, l: (b, 0, 0)),
            pl.BlockSpec((1, l_block, 1), lambda b, l: (b, l, 0)),
            pl.BlockSpec((1, l_block, nvars * D), lambda b, l: (b, l, 0)),
        ],
        out_specs=[
            pl.BlockSpec((1, l_block, nvars * D), lambda b, l: (b, l, 0)),
            pl.BlockSpec((1, l_block, nvars), lambda b, l: (b, l, 0)),
        ],
        out_shape=[
            jax.ShapeDtypeStruct((bs, L, nvars * D), xb.dtype),
            jax.ShapeDtypeStruct((bs, L, nvars), jnp.float32),
        ],
    )(nrow, ncol, x2)
    return x_masked.reshape(bs, L, nvars, D), mask


# speedup vs baseline: 1.5636x; 1.1544x over previous
import functools

import jax
import jax.numpy as jnp
from jax import lax
from jax.experimental import pallas as pl


def _mask_body(len_keep, l_block, nrow_ref, ncol_ref, xb_ref, out_ref, mask_ref):
    L = nrow_ref.shape[-1]
    nvars = mask_ref.shape[-1]
    nj = nrow_ref[0]
    nl = ncol_ref[0]
    jidx = lax.broadcasted_iota(jnp.int32, (l_block, L), 1)
    lg = pl.program_id(1) * l_block + lax.broadcasted_iota(
        jnp.int32, (l_block, L), 0)
    cnt = (nj < nl) | ((nj == nl) & (jidx < lg))
    rank = jnp.sum(cnt.astype(jnp.int32), axis=1, keepdims=True)
    keep = (rank < len_keep).astype(jnp.float32)
    out_ref[0] = xb_ref[0] * keep
    mask_ref[0] = jnp.broadcast_to(1.0 - keep, (l_block, nvars))


@jax.jit
def kernel(xb):
    bs, L, nvars, D = xb.shape
    len_keep = int(L * (1 - 0.15))
    noise = jax.random.uniform(jax.random.key(42), (bs, L), dtype=jnp.float32)
    nrow = noise.reshape(bs, 1, L)
    ncol = noise.reshape(bs, L, 1)
    x2 = xb.reshape(bs, L, nvars * D)

    l_block = 256
    grid = (bs, L // l_block)
    x_masked, mask = pl.pallas_call(
        functools.partial(_mask_body, len_keep, l_block),
        grid=grid,
        in_specs=[
            pl.BlockSpec((1, 1, L), lambda b, l: (b, 0, 0)),
            pl.BlockSpec((1, l_block, 1), lambda b, l: (b, l, 0)),
            pl.BlockSpec((1, l_block, nvars * D), lambda b, l: (b, l, 0)),
        ],
        out_specs=[
            pl.BlockSpec((1, l_block, nvars * D), lambda b, l: (b, l, 0)),
            pl.BlockSpec((1, l_block, nvars), lambda b, l: (b, l, 0)),
        ],
        out_shape=[
            jax.ShapeDtypeStruct((bs, L, nvars * D), xb.dtype),
            jax.ShapeDtypeStruct((bs, L, nvars), jnp.float32),
        ],
    )(nrow, ncol, x2)
    return x_masked.reshape(bs, L, nvars, D), mask


# speedup vs baseline: 1.5766x; 1.0083x over previous
import functools

import jax
import jax.numpy as jnp
from jax import lax
from jax.experimental import pallas as pl


def _mask_body(len_keep, l_block, nrow_ref, ncol_ref, xb_ref, out_ref, mask_ref):
    L = nrow_ref.shape[-1]
    nvars = mask_ref.shape[-1]
    nj = nrow_ref[0]
    nl = ncol_ref[0]
    jidx = lax.broadcasted_iota(jnp.int32, (l_block, L), 1)
    lg = pl.program_id(1) * l_block + lax.broadcasted_iota(
        jnp.int32, (l_block, L), 0)
    cnt = (nj < nl) | ((nj == nl) & (jidx < lg))
    rank = jnp.sum(cnt.astype(jnp.int32), axis=1, keepdims=True)
    keep = (rank < len_keep).astype(jnp.float32)
    out_ref[0] = xb_ref[0] * keep
    mask_ref[0] = jnp.broadcast_to(1.0 - keep, (l_block, nvars))


@jax.jit
def kernel(xb):
    bs, L, nvars, D = xb.shape
    len_keep = int(L * (1 - 0.15))
    noise = jax.random.uniform(jax.random.key(42), (bs, L), dtype=jnp.float32)
    nrow = noise.reshape(bs, 1, L)
    ncol = noise.reshape(bs, L, 1)
    x2 = xb.reshape(bs, L, nvars * D)

    l_block = 512
    grid = (bs, L // l_block)
    x_masked, mask = pl.pallas_call(
        functools.partial(_mask_body, len_keep, l_block),
        grid=grid,
        in_specs=[
            pl.BlockSpec((1, 1, L), lambda b, l: (b, 0, 0)),
            pl.BlockSpec((1, l_block, 1), lambda b, l: (b, l, 0)),
            pl.BlockSpec((1, l_block, nvars * D), lambda b, l: (b, l, 0)),
        ],
        out_specs=[
            pl.BlockSpec((1, l_block, nvars * D), lambda b, l: (b, l, 0)),
            pl.BlockSpec((1, l_block, nvars), lambda b, l: (b, l, 0)),
        ],
        out_shape=[
            jax.ShapeDtypeStruct((bs, L, nvars * D), xb.dtype),
            jax.ShapeDtypeStruct((bs, L, nvars), jnp.float32),
        ],
    )(nrow, ncol, x2)
    return x_masked.reshape(bs, L, nvars, D), mask


# speedup vs baseline: 2.4380x; 1.5464x over previous
import functools

import jax
import jax.numpy as jnp
from jax import lax
from jax.experimental import pallas as pl


def _mask_body(len_keep, l_block, nrow_ref, ncol_ref, xb_ref, out_ref, mask_ref):
    L = nrow_ref.shape[-1]
    nvars = mask_ref.shape[-1]
    nj = nrow_ref[0]
    nl = ncol_ref[0]
    jidx = lax.broadcasted_iota(jnp.int32, (l_block, L), 1)
    lg = pl.program_id(1) * l_block + lax.broadcasted_iota(
        jnp.int32, (l_block, L), 0)
    cnt = (nj < nl) | ((nj == nl) & (jidx < lg))
    rank = jnp.sum(cnt.astype(jnp.int32), axis=1, keepdims=True)
    keep = (rank < len_keep).astype(jnp.float32)
    out_ref[0] = xb_ref[0] * keep[:, :, None]
    mask_ref[0] = jnp.broadcast_to(1.0 - keep, (l_block, nvars))


@jax.jit
def kernel(xb):
    bs, L, nvars, D = xb.shape
    len_keep = int(L * (1 - 0.15))
    noise = jax.random.uniform(jax.random.key(42), (bs, L), dtype=jnp.float32)
    nrow = noise.reshape(bs, 1, L)
    ncol = noise.reshape(bs, L, 1)

    l_block = 128
    grid = (bs, L // l_block)
    x_masked, mask = pl.pallas_call(
        functools.partial(_mask_body, len_keep, l_block),
        grid=grid,
        in_specs=[
            pl.BlockSpec((1, 1, L), lambda b, l: (b, 0, 0)),
            pl.BlockSpec((1, l_block, 1), lambda b, l: (b, l, 0)),
            pl.BlockSpec((1, l_block, nvars, D), lambda b, l: (b, l, 0, 0)),
        ],
        out_specs=[
            pl.BlockSpec((1, l_block, nvars, D), lambda b, l: (b, l, 0, 0)),
            pl.BlockSpec((1, l_block, nvars), lambda b, l: (b, l, 0)),
        ],
        out_shape=[
            jax.ShapeDtypeStruct((bs, L, nvars, D), xb.dtype),
            jax.ShapeDtypeStruct((bs, L, nvars), jnp.float32),
        ],
    )(nrow, ncol, xb)
    return x_masked, mask


# speedup vs baseline: 2.6281x; 1.0780x over previous
import functools

import jax
import jax.numpy as jnp
from jax import lax
from jax.experimental import pallas as pl


def _mask_body(len_keep, l_block, nrow_ref, ncol_ref, xb_ref, out_ref, mask_ref):
    L = nrow_ref.shape[-1]
    nvars = mask_ref.shape[-1]
    nj = nrow_ref[0]
    nl = ncol_ref[0]
    jidx = lax.broadcasted_iota(jnp.int32, (l_block, L), 1)
    lg = pl.program_id(1) * l_block + lax.broadcasted_iota(
        jnp.int32, (l_block, L), 0)
    cnt = (nj < nl) | ((nj == nl) & (jidx < lg))
    rank = jnp.sum(cnt.astype(jnp.int32), axis=1, keepdims=True)
    keep = (rank < len_keep).astype(jnp.float32)
    out_ref[0] = xb_ref[0] * keep[:, :, None]
    mask_ref[0] = jnp.broadcast_to(1.0 - keep, (l_block, nvars))


@jax.jit
def kernel(xb):
    bs, L, nvars, D = xb.shape
    len_keep = int(L * (1 - 0.15))
    noise = jax.random.uniform(jax.random.key(42), (bs, L), dtype=jnp.float32)
    nrow = noise.reshape(bs, 1, L)
    ncol = noise.reshape(bs, L, 1)

    l_block = 256
    grid = (bs, L // l_block)
    x_masked, mask = pl.pallas_call(
        functools.partial(_mask_body, len_keep, l_block),
        grid=grid,
        in_specs=[
            pl.BlockSpec((1, 1, L), lambda b, l: (b, 0, 0)),
            pl.BlockSpec((1, l_block, 1), lambda b, l: (b, l, 0)),
            pl.BlockSpec((1, l_block, nvars, D), lambda b, l: (b, l, 0, 0)),
        ],
        out_specs=[
            pl.BlockSpec((1, l_block, nvars, D), lambda b, l: (b, l, 0, 0)),
            pl.BlockSpec((1, l_block, nvars), lambda b, l: (b, l, 0)),
        ],
        out_shape=[
            jax.ShapeDtypeStruct((bs, L, nvars, D), xb.dtype),
            jax.ShapeDtypeStruct((bs, L, nvars), jnp.float32),
        ],
    )(nrow, ncol, xb)
    return x_masked, mask


# speedup vs baseline: 2.6475x; 1.0074x over previous
import functools

import jax
import jax.numpy as jnp
from jax import lax
from jax.experimental import pallas as pl


def _mask_body(len_keep, l_block, nrow_ref, ncol_ref, xb_ref, out_ref, mask_ref):
    L = nrow_ref.shape[-1]
    nvars = mask_ref.shape[-1]
    nj = nrow_ref[0]
    nl = ncol_ref[0]
    jidx = lax.broadcasted_iota(jnp.int32, (l_block, L), 1)
    lg = pl.program_id(1) * l_block + lax.broadcasted_iota(
        jnp.int32, (l_block, L), 0)
    cnt = (nj < nl) | ((nj == nl) & (jidx < lg))
    rank = jnp.sum(cnt.astype(jnp.int32), axis=1, keepdims=True)
    keep = (rank < len_keep).astype(jnp.float32)
    out_ref[0] = xb_ref[0] * keep[:, :, None]
    mask_ref[0] = jnp.broadcast_to(1.0 - keep, (l_block, nvars))


@jax.jit
def kernel(xb):
    bs, L, nvars, D = xb.shape
    len_keep = int(L * (1 - 0.15))
    noise = jax.random.uniform(jax.random.key(42), (bs, L), dtype=jnp.float32)
    nrow = noise.reshape(bs, 1, L)
    ncol = noise.reshape(bs, L, 1)

    l_block = 512
    grid = (bs, L // l_block)
    x_masked, mask = pl.pallas_call(
        functools.partial(_mask_body, len_keep, l_block),
        grid=grid,
        in_specs=[
            pl.BlockSpec((1, 1, L), lambda b, l: (b, 0, 0)),
            pl.BlockSpec((1, l_block, 1), lambda b, l: (b, l, 0)),
            pl.BlockSpec((1, l_block, nvars, D), lambda b, l: (b, l, 0, 0)),
        ],
        out_specs=[
            pl.BlockSpec((1, l_block, nvars, D), lambda b, l: (b, l, 0, 0)),
            pl.BlockSpec((1, l_block, nvars), lambda b, l: (b, l, 0)),
        ],
        out_shape=[
            jax.ShapeDtypeStruct((bs, L, nvars, D), xb.dtype),
            jax.ShapeDtypeStruct((bs, L, nvars), jnp.float32),
        ],
    )(nrow, ncol, xb)
    return x_masked, mask


# speedup vs baseline: 2.6541x; 1.0025x over previous
import functools

import jax
import jax.numpy as jnp
from jax import lax
from jax.experimental import pallas as pl


def _mask_body(len_keep, b_block, nrow_ref, ncol_ref, xb_ref, out_ref, mask_ref):
    L = nrow_ref.shape[-1]
    nvars = mask_ref.shape[-1]
    nj = nrow_ref[...]
    nl = ncol_ref[...]
    jidx = lax.broadcasted_iota(jnp.int32, (b_block, L, L), 2)
    lg = lax.broadcasted_iota(jnp.int32, (b_block, L, L), 1)
    cnt = (nj < nl) | ((nj == nl) & (jidx < lg))
    rank = jnp.sum(cnt.astype(jnp.int32), axis=2)
    keep = (rank < len_keep).astype(jnp.float32)
    out_ref[...] = xb_ref[...] * keep[:, :, None, None]
    mask_ref[...] = jnp.broadcast_to(
        (1.0 - keep)[:, :, None], (b_block, L, nvars))


@jax.jit
def kernel(xb):
    bs, L, nvars, D = xb.shape
    len_keep = int(L * (1 - 0.15))
    noise = jax.random.uniform(jax.random.key(42), (bs, L), dtype=jnp.float32)
    nrow = noise.reshape(bs, 1, L)
    ncol = noise.reshape(bs, L, 1)

    b_block = 2
    grid = (bs // b_block,)
    x_masked, mask = pl.pallas_call(
        functools.partial(_mask_body, len_keep, b_block),
        grid=grid,
        in_specs=[
            pl.BlockSpec((b_block, 1, L), lambda b: (b, 0, 0)),
            pl.BlockSpec((b_block, L, 1), lambda b: (b, 0, 0)),
            pl.BlockSpec((b_block, L, nvars, D), lambda b: (b, 0, 0, 0)),
        ],
        out_specs=[
            pl.BlockSpec((b_block, L, nvars, D), lambda b: (b, 0, 0, 0)),
            pl.BlockSpec((b_block, L, nvars), lambda b: (b, 0, 0)),
        ],
        out_shape=[
            jax.ShapeDtypeStruct((bs, L, nvars, D), xb.dtype),
            jax.ShapeDtypeStruct((bs, L, nvars), jnp.float32),
        ],
    )(nrow, ncol, xb)
    return x_masked, mask


# speedup vs baseline: 2.7527x; 1.0372x over previous
import functools

import jax
import jax.numpy as jnp
from jax import lax
from jax.experimental import pallas as pl
from jax.experimental.pallas import tpu as pltpu

Q = 4
CH = 128


def _body(len_keep, nrow_ref, ncol_ref, xb_hbm, out_hbm, mask_ref,
          sin, sout, in_sems, out_sems):
    L = nrow_ref.shape[-1]
    nvars = mask_ref.shape[-1]
    b = pl.program_id(0)
    nb = pl.num_programs(0)
    slot = lax.rem(b, 2)
    nslot = lax.rem(b + 1, 2)

    def in_copy(bi, s, q):
        return pltpu.make_async_copy(
            xb_hbm.at[bi, pl.ds(q * CH, CH)], sin.at[s, q], in_sems.at[s, q])

    def out_copy(bi, s, q):
        return pltpu.make_async_copy(
            sout.at[s, q], out_hbm.at[bi, pl.ds(q * CH, CH)], out_sems.at[s, q])

    @pl.when(b == 0)
    def _():
        for q in range(Q):
            in_copy(0, 0, q).start()

    @pl.when(b + 1 < nb)
    def _():
        for q in range(Q):
            in_copy(b + 1, nslot, q).start()

    @pl.when(b >= 2)
    def _():
        for q in range(Q):
            out_copy(b - 2, slot, q).wait()

    nj = nrow_ref[b]
    nl = ncol_ref[b]
    jidx = lax.broadcasted_iota(jnp.int32, (L, L), 1)
    lg = lax.broadcasted_iota(jnp.int32, (L, L), 0)
    cnt = (nj < nl) | ((nj == nl) & (jidx < lg))
    rank = jnp.sum(cnt.astype(jnp.int32), axis=1, keepdims=True)
    keep = (rank < len_keep).astype(jnp.float32)

    for q in range(Q):
        in_copy(b, slot, q).wait()
        kq = keep[q * CH:(q + 1) * CH]
        sout[slot, q] = sin[slot, q] * kq[:, :, None]
        out_copy(b, slot, q).start()

    mask_ref[0] = jnp.broadcast_to(1.0 - keep, (L, nvars))

    @pl.when(b == nb - 1)
    def _():
        for q in range(Q):
            out_copy(b - 1, nslot, q).wait()
            out_copy(b, slot, q).wait()


@jax.jit
def kernel(xb):
    bs, L, nvars, D = xb.shape
    len_keep = int(L * (1 - 0.15))
    noise = jax.random.uniform(jax.random.key(42), (bs, L), dtype=jnp.float32)
    nrow = noise.reshape(bs, 1, L)
    ncol = noise.reshape(bs, L, 1)

    x_masked, mask = pl.pallas_call(
        functools.partial(_body, len_keep),
        grid=(bs,),
        in_specs=[
            pl.BlockSpec((bs, 1, L), lambda b: (0, 0, 0)),
            pl.BlockSpec((bs, L, 1), lambda b: (0, 0, 0)),
            pl.BlockSpec(memory_space=pl.ANY),
        ],
        out_specs=[
            pl.BlockSpec(memory_space=pl.ANY),
            pl.BlockSpec((1, L, nvars), lambda b: (b, 0, 0)),
        ],
        out_shape=[
            jax.ShapeDtypeStruct((bs, L, nvars, D), xb.dtype),
            jax.ShapeDtypeStruct((bs, L, nvars), jnp.float32),
        ],
        scratch_shapes=[
            pltpu.VMEM((2, Q, CH, nvars, D), jnp.float32),
            pltpu.VMEM((2, Q, CH, nvars, D), jnp.float32),
            pltpu.SemaphoreType.DMA((2, Q)),
            pltpu.SemaphoreType.DMA((2, Q)),
        ],
    )(nrow, ncol, xb)
    return x_masked, mask
